# baseline (speedup 1.0000x reference)
import functools
import jax
import jax.numpy as jnp
from jax import lax
from jax.experimental import pallas as pl
from jax.experimental.pallas import tpu as pltpu
from jax.experimental.pallas import tpu_sc as plsc

N = 20000
G = 100
NS = 16
PW = N // NS
APW = ((PW + 15) // 16) * 16
NCH = APW // 16
NPAD = NS * APW
GPAD = 128
TOPK = 9
NEG_INF = -100000000.0
BIG = 1e30
PADV = 4e8


def _newton_sqrt(v):
    bits = plsc.bitcast(v, jnp.int32)
    y = plsc.bitcast((bits >> 1) + jnp.int32(0x1FBD1DF5), jnp.float32)
    for _ in range(4):
        y = 0.5 * (y + v / y)
    return jnp.where(v > 0.0, y, 0.0)


def _sc_body(ax0_h, ay0_h, ax1_h, ay1_h, gx0_h, gy0_h, gx1_h, gy1_h,
             out_h,
             a_x0, a_y0, a_x1, a_y1, acx, acy,
             g_x0, g_y0, g_x1, g_y1,
             l_d, l_i, l_iou, l_cx, l_cy,
             t_d, t_i, t_iou, t_cx, t_cy,
             c_i, c_v, b_v, b_g, outv,
             sh_d, sh_i, sh_iou, sh_cx, sh_cy, sh_ci, sh_cv,
             sem):
    w = lax.axis_index("s")
    base = w * APW
    iota = lax.iota(jnp.int32, 16)

    pltpu.sync_copy(ax0_h.at[pl.ds(base, APW)], a_x0)
    pltpu.sync_copy(ay0_h.at[pl.ds(base, APW)], a_y0)
    pltpu.sync_copy(ax1_h.at[pl.ds(base, APW)], a_x1)
    pltpu.sync_copy(ay1_h.at[pl.ds(base, APW)], a_y1)
    pltpu.sync_copy(gx0_h, g_x0)
    pltpu.sync_copy(gy0_h, g_y0)
    pltpu.sync_copy(gx1_h, g_x1)
    pltpu.sync_copy(gy1_h, g_y1)

    def centers(c, _):
        off = pl.multiple_of(c * 16, 16)
        acx[pl.ds(off, 16)] = (a_x1[pl.ds(off, 16)] + a_x0[pl.ds(off, 16)]) / 2.0
        acy[pl.ds(off, 16)] = (a_y1[pl.ds(off, 16)] + a_y0[pl.ds(off, 16)]) / 2.0
        return 0
    lax.fori_loop(0, NCH, centers, 0)

    def per_gt(g, _):
        sg = jnp.full((16,), g, jnp.int32)
        gx0s = plsc.load_gather(g_x0, [sg])
        gy0s = plsc.load_gather(g_y0, [sg])
        gx1s = plsc.load_gather(g_x1, [sg])
        gy1s = plsc.load_gather(g_y1, [sg])
        gcx = (gx1s + gx0s) / 2.0
        gcy = (gy1s + gy0s) / 2.0

        def chunk(c, carry):
            bd, bi = carry
            off = pl.multiple_of(c * 16, 16)
            dx = acx[pl.ds(off, 16)] - gcx
            dy = acy[pl.ds(off, 16)] - gcy
            d2 = dx * dx + dy * dy
            thr = bd[8]
            mn = jnp.min(d2)

            def merge(args):
                bd, bi = args
                gi = iota + (base + off)
                sd, si = plsc.sort_key_val(d2, gi)
                rd = lax.rev(sd, (0,))
                ri = lax.rev(si, (0,))
                take = bd <= rd
                md = jnp.where(take, bd, rd)
                mi = jnp.where(take, bi, ri)
                return tuple(plsc.sort_key_val(md, mi))

            return lax.cond(mn < thr, merge, lambda a: a, (bd, bi))

        bd0 = jnp.full((16,), BIG, jnp.float32)
        bi0 = jnp.zeros((16,), jnp.int32)
        bd, bi = lax.fori_loop(0, NCH, chunk, (bd0, bi0))

        li = bi - jnp.full((16,), base, jnp.int32)
        x0 = plsc.load_gather(a_x0, [li])
        y0 = plsc.load_gather(a_y0, [li])
        x1 = plsc.load_gather(a_x1, [li])
        y1 = plsc.load_gather(a_y1, [li])
        cx = (x1 + x0) / 2.0
        cy = (y1 + y0) / 2.0
        area_a = (x1 - x0) * (y1 - y0)
        area_g = (gx1s - gx0s) * (gy1s - gy0s)
        iw = jnp.maximum(jnp.minimum(x1, gx1s) - jnp.maximum(x0, gx0s), 0.0)
        ih = jnp.maximum(jnp.minimum(y1, gy1s) - jnp.maximum(y0, gy0s), 0.0)
        inter = iw * ih
        iou = inter / (area_a + area_g - inter)

        row = pl.ds(pl.multiple_of(g * 16, 16), 16)
        l_d[row] = bd
        l_i[row] = bi
        l_iou[row] = iou
        l_cx[row] = cx
        l_cy[row] = cy
        return 0
    lax.fori_loop(0, G, per_gt, 0)

    wrow = pl.ds(w * (G * 16), G * 16)
    pltpu.sync_copy(l_d, sh_d.at[wrow])
    pltpu.sync_copy(l_i, sh_i.at[wrow])
    pltpu.sync_copy(l_iou, sh_iou.at[wrow])
    pltpu.sync_copy(l_cx, sh_cx.at[wrow])
    pltpu.sync_copy(l_cy, sh_cy.at[wrow])
    plsc.subcore_barrier()

    q, rem = divmod(G, NS)
    gstart = w * q + jnp.minimum(w, rem)
    gcount = q + jnp.where(w < rem, 1, 0)

    def own_gt(g, _):
        descs = []
        for r in range(NS):
            src = pl.ds(pl.multiple_of(r * (G * 16) + g * 16, 16), 16)
            dst = pl.ds(r * 16, 16)
            descs.append(pltpu.async_copy(sh_d.at[src], t_d.at[dst], sem))
            descs.append(pltpu.async_copy(sh_i.at[src], t_i.at[dst], sem))
            descs.append(pltpu.async_copy(sh_iou.at[src], t_iou.at[dst], sem))
            descs.append(pltpu.async_copy(sh_cx.at[src], t_cx.at[dst], sem))
            descs.append(pltpu.async_copy(sh_cy.at[src], t_cy.at[dst], sem))
        for d in descs:
            d.wait()

        acc_d = t_d[pl.ds(0, 16)]
        acc_p = iota

        def mrow(r, carry):
            acc_d, acc_p = carry
            ch_d = t_d[pl.ds(pl.multiple_of(r * 16, 16), 16)]
            ch_p = r * 16 + iota
            rd = lax.rev(ch_d, (0,))
            rp = lax.rev(ch_p, (0,))
            take = acc_d <= rd
            md = jnp.where(take, acc_d, rd)
            mp = jnp.where(take, acc_p, rp)
            return tuple(plsc.sort_key_val(md, mp))
        acc_d, acc_p = lax.fori_loop(1, NS, mrow, (acc_d, acc_p))

        idx9 = plsc.load_gather(t_i, [acc_p])
        iou9 = plsc.load_gather(t_iou, [acc_p])
        cx9 = plsc.load_gather(t_cx, [acc_p])
        cy9 = plsc.load_gather(t_cy, [acc_p])

        m9 = iota < TOPK
        sg = jnp.full((16,), g, jnp.int32)
        gx0s = plsc.load_gather(g_x0, [sg])
        gy0s = plsc.load_gather(g_y0, [sg])
        gx1s = plsc.load_gather(g_x1, [sg])
        gy1s = plsc.load_gather(g_y1, [sg])

        mean = jnp.full((16,), jnp.sum(jnp.where(m9, iou9, 0.0)),
                        jnp.float32) / float(TOPK)
        dv = jnp.where(m9, iou9 - mean, 0.0)
        var = jnp.full((16,), jnp.sum(dv * dv), jnp.float32) / float(TOPK - 1)
        thresh = mean + _newton_sqrt(var)

        inbox = jnp.minimum(
            jnp.minimum(cx9 - gx0s, cy9 - gy0s),
            jnp.minimum(gx1s - cx9, gy1s - cy9)) > 0.01
        pos = m9 & (iou9 >= thresh) & inbox
        cidx = jnp.where(pos, idx9, -1)

        t_i[pl.ds(0, 16)] = cidx
        t_iou[pl.ds(0, 16)] = iou9
        gout = pl.ds(pl.multiple_of(g * 16, 16), 16)
        pltpu.sync_copy(t_i.at[pl.ds(0, 16)], sh_ci.at[gout])
        pltpu.sync_copy(t_iou.at[pl.ds(0, 16)], sh_cv.at[gout])
        return 0
    lax.fori_loop(gstart, gstart + gcount, own_gt, 0)
    plsc.subcore_barrier()

    pltpu.sync_copy(sh_ci, c_i)
    pltpu.sync_copy(sh_cv, c_v)

    def init(c, _):
        off = pl.multiple_of(c * 16, 16)
        b_v[pl.ds(off, 16)] = jnp.full((16,), NEG_INF, jnp.float32)
        b_g[pl.ds(off, 16)] = jnp.zeros((16,), jnp.int32)
        return 0
    lax.fori_loop(0, NCH, init, 0)

    basev = jnp.full((16,), base, jnp.int32)

    def assign(g, _):
        row = pl.ds(pl.multiple_of(g * 16, 16), 16)
        ci = c_i[row]
        cv = c_v[row]
        li = ci - basev
        inr = (ci >= 0) & (li >= 0) & (li < APW)
        safe = jnp.where(inr, li, 0)
        cur = plsc.load_gather(b_v, [safe])
        upd = inr & (cv > cur)
        plsc.store_scatter(b_v, [safe], cv, mask=upd)
        plsc.store_scatter(b_g, [safe], jnp.full((16,), g, jnp.int32),
                           mask=upd)
        return 0
    lax.fori_loop(0, G, assign, 0)

    def emit(c, _):
        off = pl.multiple_of(c * 16, 16)
        bg = b_g[pl.ds(off, 16)]
        bv = b_v[pl.ds(off, 16)]
        x0 = plsc.load_gather(g_x0, [bg])
        y0 = plsc.load_gather(g_y0, [bg])
        x1 = plsc.load_gather(g_x1, [bg])
        y1 = plsc.load_gather(g_y1, [bg])
        p = (c * 16 + iota) * 5
        plsc.store_scatter(outv, [p], x0)
        plsc.store_scatter(outv, [p + 1], y0)
        plsc.store_scatter(outv, [p + 2], x1)
        plsc.store_scatter(outv, [p + 3], y1)
        plsc.store_scatter(outv, [p + 4], bv)
        return 0
    lax.fori_loop(0, NCH, emit, 0)
    pltpu.sync_copy(outv, out_h.at[pl.ds(w * (APW * 5), APW * 5)])


def _make_sc_call():
    mesh = plsc.VectorSubcoreMesh(core_axis_name="c", subcore_axis_name="s",
                                  num_cores=1, num_subcores=NS)
    return functools.partial(
        pl.kernel, mesh=mesh,
        out_type=jax.ShapeDtypeStruct((NPAD * 5,), jnp.float32),
        scratch_types=[
            pltpu.VMEM((APW,), jnp.float32), pltpu.VMEM((APW,), jnp.float32),
            pltpu.VMEM((APW,), jnp.float32), pltpu.VMEM((APW,), jnp.float32),
            pltpu.VMEM((APW,), jnp.float32), pltpu.VMEM((APW,), jnp.float32),
            pltpu.VMEM((GPAD,), jnp.float32), pltpu.VMEM((GPAD,), jnp.float32),
            pltpu.VMEM((GPAD,), jnp.float32), pltpu.VMEM((GPAD,), jnp.float32),
            pltpu.VMEM((G * 16,), jnp.float32), pltpu.VMEM((G * 16,), jnp.int32),
            pltpu.VMEM((G * 16,), jnp.float32), pltpu.VMEM((G * 16,), jnp.float32),
            pltpu.VMEM((G * 16,), jnp.float32),
            pltpu.VMEM((NS * 16,), jnp.float32), pltpu.VMEM((NS * 16,), jnp.int32),
            pltpu.VMEM((NS * 16,), jnp.float32), pltpu.VMEM((NS * 16,), jnp.float32),
            pltpu.VMEM((NS * 16,), jnp.float32),
            pltpu.VMEM((G * 16,), jnp.int32), pltpu.VMEM((G * 16,), jnp.float32),
            pltpu.VMEM((APW,), jnp.float32), pltpu.VMEM((APW,), jnp.int32),
            pltpu.VMEM((APW * 5,), jnp.float32),
            pltpu.VMEM_SHARED((NS * G * 16,), jnp.float32),
            pltpu.VMEM_SHARED((NS * G * 16,), jnp.int32),
            pltpu.VMEM_SHARED((NS * G * 16,), jnp.float32),
            pltpu.VMEM_SHARED((NS * G * 16,), jnp.float32),
            pltpu.VMEM_SHARED((NS * G * 16,), jnp.float32),
            pltpu.VMEM_SHARED((G * 16,), jnp.int32),
            pltpu.VMEM_SHARED((G * 16,), jnp.float32),
            pltpu.SemaphoreType.DMA,
        ],
        compiler_params=pltpu.CompilerParams(needs_layout_passes=False),
    )(_sc_body)


def _pad_anchor_col(col):
    return jnp.concatenate(
        [col.reshape(NS, PW),
         jnp.full((NS, APW - PW), PADV, jnp.float32)], axis=1).reshape(-1)


def kernel(pred_boxes, targets):
    anchors = pred_boxes[0]
    bboxes = targets[:, 1:-1]
    ax0 = _pad_anchor_col(anchors[:, 0])
    ay0 = _pad_anchor_col(anchors[:, 1])
    ax1 = _pad_anchor_col(anchors[:, 2])
    ay1 = _pad_anchor_col(anchors[:, 3])
    gpad = jnp.zeros((GPAD - G,), jnp.float32)
    gx0 = jnp.concatenate([bboxes[:, 0], gpad])
    gy0 = jnp.concatenate([bboxes[:, 1], gpad])
    gx1 = jnp.concatenate([bboxes[:, 2], gpad])
    gy1 = jnp.concatenate([bboxes[:, 3], gpad])
    out = _make_sc_call()(ax0, ay0, ax1, ay1, gx0, gy0, gx1, gy1)
    return out.reshape(NS, APW, 5)[:, :PW].reshape(N, 5)
